# Initial kernel scaffold; baseline (speedup 1.0000x reference)
import jax
import jax.numpy as jnp
from jax.experimental import pallas as pl


def kernel(x, edge_index, W1, b1, W2, b2, temp):
    raise NotImplementedError("write your pallas kernel here")


# speedup vs baseline: 77.4071x; 77.4071x over previous
import functools
import math

import jax
import jax.numpy as jnp
import numpy as np
from jax import lax
from jax.experimental import pallas as pl
from jax.experimental.pallas import tpu as pltpu
from jax.experimental.pallas import tpu_sc as plsc

N = 10000
E = 320000
D = 128
HID = 64
C = 16
K = 10

NC = 2
NS = 16
NW = NC * NS
CH = 128
N_PAD = 10240
PER_TILE = E // NW
NCH = 80
PER_TILE_PAD = NCH * CH
E_PAD = NW * PER_TILE_PAD
ROWS_PER_TILE = N_PAD // NS


def _coef_matrix():
    u = np.zeros((K + 1, K + 1), dtype=np.float64)
    for j in range(K + 1):
        for a in range(j + 1):
            for b in range(K - j + 1):
                u[j, a + b] += math.comb(j, a) * math.comb(K - j, b) * ((-1.0) ** b)
    return u.astype(np.float32)


_U = _coef_matrix()
_BINOM = np.asarray([math.comb(K, j) / 2.0**K for j in range(K + 1)],
                    dtype=np.float32)

@functools.cache
def _get_sc_scatter_add():
    mesh = plsc.VectorSubcoreMesh(core_axis_name="c", subcore_axis_name="s")

    @functools.partial(
        pl.kernel,
        mesh=mesh,
        compiler_params=pltpu.CompilerParams(use_tc_tiling_on_sc=False),
        out_type=jax.ShapeDtypeStruct((NC, N_PAD, C), jnp.float32),
        scratch_types=[
            pltpu.VMEM((NCH, CH), jnp.int32),
            pltpu.VMEM((NCH, CH), jnp.int32),
            pltpu.VMEM((2, CH, C), jnp.float32),
            pltpu.VMEM((ROWS_PER_TILE, C), jnp.float32),
            pltpu.VMEM_SHARED((N_PAD, C), jnp.float32),
        ],
    )
    def _sc_scatter_add(table, src, dst, zeros, out, src_v, dst_v, rows_v,
                        stage_v, acc):
        cid = lax.axis_index("c")
        sid = lax.axis_index("s")
        wid = cid * NS + sid
        pltpu.sync_copy(src.at[wid], src_v)
        pltpu.sync_copy(dst.at[wid], dst_v)
        r0 = sid * ROWS_PER_TILE
        pltpu.sync_copy(zeros.at[pl.ds(r0, ROWS_PER_TILE)], stage_v)
        pltpu.sync_copy(stage_v, acc.at[pl.ds(r0, ROWS_PER_TILE)])
        plsc.subcore_barrier()

        def chunk(j, buf):
            pltpu.sync_copy(table.at[src_v.at[j]], rows_v.at[buf])
            pltpu.sync_copy(rows_v.at[buf], acc.at[dst_v.at[j]], add=True)

        def body(j2, carry):
            chunk(2 * j2, 0)
            chunk(2 * j2 + 1, 1)
            return carry

        lax.fori_loop(0, NCH // 2, body, 0)
        plsc.subcore_barrier()
        pltpu.sync_copy(acc.at[pl.ds(r0, ROWS_PER_TILE)], stage_v)
        pltpu.sync_copy(stage_v, out.at[cid, pl.ds(r0, ROWS_PER_TILE)])

    return _sc_scatter_add


def _mlp_body(x_ref, w1_ref, b1_ref, w2_ref, b2_ref, h_ref):
    h1 = jnp.dot(x_ref[...], w1_ref[...], preferred_element_type=jnp.float32)
    h1 = jnp.maximum(h1 + b1_ref[...], 0.0)
    h_ref[...] = (jnp.dot(h1, w2_ref[...], preferred_element_type=jnp.float32)
                  + b2_ref[...])


def _init_body(degp_ref, h_ref, dk_ref, u_ref, dis_ref):
    deg = degp_ref[0] + degp_ref[1]
    dis = jnp.where(deg > 0, lax.rsqrt(deg), 0.0)
    dis_ref[...] = dis
    u_ref[...] = dis * (dk_ref[0, 0] * h_ref[...])


def _combine_body(y_ref, dis_ref, h_ref, dp_ref, u_ref):
    dis = dis_ref[...]
    r = dp_ref[0, 0] * h_ref[...] - dis * (y_ref[0] + y_ref[1])
    u_ref[...] = dis * r


def _final_body(y_ref, dis_ref, h_ref, d0_ref, o_ref):
    r = d0_ref[0, 0] * h_ref[...] - dis_ref[...] * (y_ref[0] + y_ref[1])
    m = jnp.max(r, axis=1, keepdims=True)
    o_ref[...] = (r - m) - jnp.log(jnp.sum(jnp.exp(r - m), axis=1,
                                           keepdims=True))


_VMEM = pl.BlockSpec(memory_space=pltpu.VMEM)
_SMEM = pl.BlockSpec(memory_space=pltpu.SMEM)


def _shape(s):
    return jax.ShapeDtypeStruct(s, jnp.float32)


_mlp = pl.pallas_call(
    _mlp_body,
    in_specs=[_VMEM] * 5,
    out_specs=_VMEM,
    out_shape=_shape((N_PAD, C)),
)
_init = pl.pallas_call(
    _init_body,
    in_specs=[_VMEM, _VMEM, _SMEM],
    out_specs=(_VMEM, _VMEM),
    out_shape=(_shape((N_PAD, C)), _shape((N_PAD, C))),
)
_combine = pl.pallas_call(
    _combine_body,
    in_specs=[_VMEM, _VMEM, _VMEM, _SMEM],
    out_specs=_VMEM,
    out_shape=_shape((N_PAD, C)),
)
_final = pl.pallas_call(
    _final_body,
    in_specs=[_VMEM, _VMEM, _VMEM, _SMEM],
    out_specs=_VMEM,
    out_shape=_shape((N_PAD, C)),
)


def kernel(x, edge_index, W1, b1, W2, b2, temp):
    f32 = jnp.float32
    row = edge_index[0]
    col = edge_index[1]
    npad = E_PAD - E
    pad_r = (jnp.int32(N) + jnp.arange(npad, dtype=jnp.int32) % (N_PAD - N))
    pad_c = (jnp.int32(N)
             + (jnp.arange(npad, dtype=jnp.int32) * 7 + 3) % (N_PAD - N))
    src_arr = jnp.concatenate([row, pad_r]).reshape(NW, NCH, CH)
    dst_arr = jnp.concatenate([col, pad_c]).reshape(NW, NCH, CH)

    zeros_t = jnp.zeros((N_PAD, C), f32)
    ones_t = jnp.ones((N_PAD, C), f32)
    x_pad = jnp.zeros((N_PAD, D), f32).at[:N].set(x)

    h = _mlp(x_pad, W1, b1.reshape(1, HID), W2, b2.reshape(1, C))
    dvec = (jax.nn.relu(temp) * _BINOM) @ _U

    sc_scatter_add = _get_sc_scatter_add()
    degp = sc_scatter_add(ones_t, dst_arr, src_arr, zeros_t)
    u, dis = _init(degp, h, dvec[K].reshape(1, 1))

    for p in range(K - 1, 0, -1):
        y = sc_scatter_add(u, src_arr, dst_arr, zeros_t)
        u = _combine(y, dis, h, dvec[p].reshape(1, 1))
    y = sc_scatter_add(u, src_arr, dst_arr, zeros_t)
    out = _final(y, dis, h, dvec[0].reshape(1, 1))
    return out[:N]


# speedup vs baseline: 133.9766x; 1.7308x over previous
import functools
import math

import jax
import jax.numpy as jnp
import numpy as np
from jax import lax
from jax.experimental import pallas as pl
from jax.experimental.pallas import tpu as pltpu
from jax.experimental.pallas import tpu_sc as plsc

N = 10000
E = 320000
D = 128
HID = 64
C = 16
K = 10

NC = 2
NS = 16
NW = NC * NS
CH = 128
N_PAD = 10240
PER_TILE = E // NW
NCH = 80
NBUF = 4
NOV = NBUF
NCH_TOT = NCH + NOV
E_SCAT = NW * NCH * CH
E_TOT = NW * NCH_TOT * CH
ROWS_PER_TILE = N_PAD // NS


def _coef_matrix():
    u = np.zeros((K + 1, K + 1), dtype=np.float64)
    for j in range(K + 1):
        for a in range(j + 1):
            for b in range(K - j + 1):
                u[j, a + b] += math.comb(j, a) * math.comb(K - j, b) * ((-1.0) ** b)
    return u.astype(np.float32)


_U = _coef_matrix()
_BINOM = np.asarray([math.comb(K, j) / 2.0**K for j in range(K + 1)],
                    dtype=np.float32)

@functools.cache
def _get_sc_scatter_add():
    mesh = plsc.VectorSubcoreMesh(core_axis_name="c", subcore_axis_name="s")

    @functools.partial(
        pl.kernel,
        mesh=mesh,
        compiler_params=pltpu.CompilerParams(use_tc_tiling_on_sc=False),
        out_type=jax.ShapeDtypeStruct((NC, N_PAD, C), jnp.float32),
        scratch_types=[
            pltpu.VMEM((NCH_TOT, CH), jnp.int32),
            pltpu.VMEM((NCH_TOT, CH), jnp.int32),
            pltpu.VMEM((2, NBUF, CH, C), jnp.float32),
            pltpu.VMEM((ROWS_PER_TILE, C), jnp.float32),
            pltpu.VMEM_SHARED((N_PAD, C), jnp.float32),
            pltpu.SemaphoreType.DMA,
            pltpu.SemaphoreType.DMA,
        ],
    )
    def _sc_scatter_add(table, src, dst, zeros, out, src_v, dst_v, rows_v,
                        stage_v, acc, gsem, ssem):
        cid = lax.axis_index("c")
        sid = lax.axis_index("s")
        wid = cid * NS + sid
        pltpu.sync_copy(src.at[wid], src_v)
        pltpu.sync_copy(dst.at[wid], dst_v)
        r0 = sid * ROWS_PER_TILE
        pltpu.sync_copy(zeros.at[pl.ds(r0, ROWS_PER_TILE)], stage_v)
        pltpu.sync_copy(stage_v, acc.at[pl.ds(r0, ROWS_PER_TILE)])
        plsc.subcore_barrier()

        def issue_gathers(g, s):
            for b in range(NBUF):
                pltpu.async_copy(table.at[src_v.at[g * NBUF + b]],
                                 rows_v.at[s, b], gsem)

        def drain_gathers(s):
            for b in range(NBUF):
                pltpu.make_async_copy(table.at[src_v.at[0]],
                                      rows_v.at[s, b], gsem).wait()

        def issue_scatters(g, s):
            for b in range(NBUF):
                pltpu.async_copy(rows_v.at[s, b],
                                 acc.at[dst_v.at[g * NBUF + b]], ssem,
                                 add=True)

        def drain_scatters(s):
            for b in range(NBUF):
                pltpu.make_async_copy(rows_v.at[s, b],
                                      acc.at[dst_v.at[0]], ssem).wait()

        issue_gathers(0, 0)

        def body(g2, carry):
            ga = 2 * g2
            gb = 2 * g2 + 1
            drain_gathers(0)
            issue_gathers(gb, 1)
            issue_scatters(ga, 0)
            drain_gathers(1)
            drain_scatters(0)
            issue_gathers(ga + 2, 0)
            issue_scatters(gb, 1)
            drain_scatters(1)
            return carry

        lax.fori_loop(0, NCH // (2 * NBUF), body, 0)
        drain_gathers(0)
        plsc.subcore_barrier()
        pltpu.sync_copy(acc.at[pl.ds(r0, ROWS_PER_TILE)], stage_v)
        pltpu.sync_copy(stage_v, out.at[cid, pl.ds(r0, ROWS_PER_TILE)])

    return _sc_scatter_add


def _mlp_body(x_ref, w1_ref, b1_ref, w2_ref, b2_ref, h_ref):
    h1 = jnp.dot(x_ref[...], w1_ref[...], preferred_element_type=jnp.float32)
    h1 = jnp.maximum(h1 + b1_ref[...], 0.0)
    h_ref[...] = (jnp.dot(h1, w2_ref[...], preferred_element_type=jnp.float32)
                  + b2_ref[...])


def _init_body(degp_ref, h_ref, dk_ref, u_ref, dis_ref):
    deg = degp_ref[0] + degp_ref[1]
    dis = jnp.where(deg > 0, lax.rsqrt(deg), 0.0)
    dis_ref[...] = dis
    u_ref[...] = dis * (dk_ref[0, 0] * h_ref[...])


def _combine_body(y_ref, dis_ref, h_ref, dp_ref, u_ref):
    dis = dis_ref[...]
    r = dp_ref[0, 0] * h_ref[...] - dis * (y_ref[0] + y_ref[1])
    u_ref[...] = dis * r


def _final_body(y_ref, dis_ref, h_ref, d0_ref, o_ref):
    r = d0_ref[0, 0] * h_ref[...] - dis_ref[...] * (y_ref[0] + y_ref[1])
    m = jnp.max(r, axis=1, keepdims=True)
    o_ref[...] = (r - m) - jnp.log(jnp.sum(jnp.exp(r - m), axis=1,
                                           keepdims=True))


_VMEM = pl.BlockSpec(memory_space=pltpu.VMEM)
_SMEM = pl.BlockSpec(memory_space=pltpu.SMEM)


def _shape(s):
    return jax.ShapeDtypeStruct(s, jnp.float32)


_mlp = pl.pallas_call(
    _mlp_body,
    in_specs=[_VMEM] * 5,
    out_specs=_VMEM,
    out_shape=_shape((N_PAD, C)),
)
_init = pl.pallas_call(
    _init_body,
    in_specs=[_VMEM, _VMEM, _SMEM],
    out_specs=(_VMEM, _VMEM),
    out_shape=(_shape((N_PAD, C)), _shape((N_PAD, C))),
)
_combine = pl.pallas_call(
    _combine_body,
    in_specs=[_VMEM, _VMEM, _VMEM, _SMEM],
    out_specs=_VMEM,
    out_shape=_shape((N_PAD, C)),
)
_final = pl.pallas_call(
    _final_body,
    in_specs=[_VMEM, _VMEM, _VMEM, _SMEM],
    out_specs=_VMEM,
    out_shape=_shape((N_PAD, C)),
)


def kernel(x, edge_index, W1, b1, W2, b2, temp):
    f32 = jnp.float32
    row = edge_index[0]
    col = edge_index[1]
    npad = E_SCAT - E
    pad_r = (jnp.int32(N) + jnp.arange(npad, dtype=jnp.int32) % (N_PAD - N))
    pad_c = (jnp.int32(N)
             + (jnp.arange(npad, dtype=jnp.int32) * 7 + 3) % (N_PAD - N))
    nov = E_TOT - E_SCAT
    ov = ((jnp.arange(nov, dtype=jnp.int32) * 97) % N_PAD).reshape(NW, NOV, CH)
    src_arr = jnp.concatenate(
        [jnp.concatenate([row, pad_r]).reshape(NW, NCH, CH), ov], axis=1)
    dst_arr = jnp.concatenate(
        [jnp.concatenate([col, pad_c]).reshape(NW, NCH, CH), ov], axis=1)

    zeros_t = jnp.zeros((N_PAD, C), f32)
    ones_t = jnp.ones((N_PAD, C), f32)
    x_pad = jnp.zeros((N_PAD, D), f32).at[:N].set(x)

    h = _mlp(x_pad, W1, b1.reshape(1, HID), W2, b2.reshape(1, C))
    dvec = (jax.nn.relu(temp) * _BINOM) @ _U

    sc_scatter_add = _get_sc_scatter_add()
    degp = sc_scatter_add(ones_t, dst_arr, src_arr, zeros_t)
    u, dis = _init(degp, h, dvec[K].reshape(1, 1))

    for p in range(K - 1, 0, -1):
        y = sc_scatter_add(u, src_arr, dst_arr, zeros_t)
        u = _combine(y, dis, h, dvec[p].reshape(1, 1))
    y = sc_scatter_add(u, src_arr, dst_arr, zeros_t)
    out = _final(y, dis, h, dvec[0].reshape(1, 1))
    return out[:N]


# speedup vs baseline: 202.9630x; 1.5149x over previous
import functools
import math

import jax
import jax.numpy as jnp
import numpy as np
from jax import lax
from jax.experimental import pallas as pl
from jax.experimental.pallas import tpu as pltpu
from jax.experimental.pallas import tpu_sc as plsc

N = 10000
E = 320000
D = 128
HID = 64
C = 16
K = 10

NC = 2
NS = 16
NW = NC * NS
CH = 128
N_PAD = 10240
PER_TILE = E // NW
NCH = 80
NBUF = 5
NOV = NBUF
NCH_TOT = NCH + NOV
E_SCAT = NW * NCH * CH
E_TOT = NW * NCH_TOT * CH
ROWS_PER_TILE = N_PAD // NS


def _coef_matrix():
    u = np.zeros((K + 1, K + 1), dtype=np.float64)
    for j in range(K + 1):
        for a in range(j + 1):
            for b in range(K - j + 1):
                u[j, a + b] += math.comb(j, a) * math.comb(K - j, b) * ((-1.0) ** b)
    return u.astype(np.float32)


_U = _coef_matrix()
_BINOM = np.asarray([math.comb(K, j) / 2.0**K for j in range(K + 1)],
                    dtype=np.float32)

@functools.cache
def _get_sc_scatter_add():
    mesh = plsc.VectorSubcoreMesh(core_axis_name="c", subcore_axis_name="s")

    @functools.partial(
        pl.kernel,
        mesh=mesh,
        compiler_params=pltpu.CompilerParams(use_tc_tiling_on_sc=False),
        out_type=jax.ShapeDtypeStruct((NC, N_PAD, C), jnp.float32),
        scratch_types=[
            pltpu.VMEM((NCH_TOT, CH), jnp.int32),
            pltpu.VMEM((NCH_TOT, CH), jnp.int32),
            pltpu.VMEM((2, NBUF, CH, C), jnp.float32),
            pltpu.VMEM((ROWS_PER_TILE, C), jnp.float32),
            pltpu.VMEM_SHARED((N_PAD, C), jnp.float32),
            pltpu.SemaphoreType.DMA,
            pltpu.SemaphoreType.DMA,
        ],
    )
    def _sc_scatter_add(table, src, dst, zeros, out, src_v, dst_v, rows_v,
                        stage_v, acc, gsem, ssem):
        cid = lax.axis_index("c")
        sid = lax.axis_index("s")
        wid = cid * NS + sid
        pltpu.sync_copy(src.at[wid], src_v)
        pltpu.sync_copy(dst.at[wid], dst_v)
        r0 = sid * ROWS_PER_TILE
        pltpu.sync_copy(zeros.at[pl.ds(r0, ROWS_PER_TILE)], stage_v)
        pltpu.sync_copy(stage_v, acc.at[pl.ds(r0, ROWS_PER_TILE)])
        plsc.subcore_barrier()

        def issue_gathers(g, s):
            for b in range(NBUF):
                pltpu.async_copy(table.at[src_v.at[g * NBUF + b]],
                                 rows_v.at[s, b], gsem)

        def drain_gathers(s):
            for b in range(NBUF):
                pltpu.make_async_copy(table.at[src_v.at[0]],
                                      rows_v.at[s, b], gsem).wait()

        def issue_scatters(g, s):
            for b in range(NBUF):
                pltpu.async_copy(rows_v.at[s, b],
                                 acc.at[dst_v.at[g * NBUF + b]], ssem,
                                 add=True)

        def drain_scatters(s):
            for b in range(NBUF):
                pltpu.make_async_copy(rows_v.at[s, b],
                                      acc.at[dst_v.at[0]], ssem).wait()

        issue_gathers(0, 0)

        def body(g2, carry):
            ga = 2 * g2
            gb = 2 * g2 + 1
            drain_gathers(0)
            issue_gathers(gb, 1)
            issue_scatters(ga, 0)
            drain_gathers(1)
            drain_scatters(0)
            issue_gathers(ga + 2, 0)
            issue_scatters(gb, 1)
            drain_scatters(1)
            return carry

        lax.fori_loop(0, NCH // (2 * NBUF), body, 0)
        drain_gathers(0)
        plsc.subcore_barrier()
        pltpu.sync_copy(acc.at[pl.ds(r0, ROWS_PER_TILE)], stage_v)
        pltpu.sync_copy(stage_v, out.at[cid, pl.ds(r0, ROWS_PER_TILE)])

    return _sc_scatter_add


def _mlp_body(x_ref, w1_ref, b1_ref, w2_ref, b2_ref, h_ref):
    h1 = jnp.dot(x_ref[...], w1_ref[...], preferred_element_type=jnp.float32)
    h1 = jnp.maximum(h1 + b1_ref[...], 0.0)
    h_ref[...] = (jnp.dot(h1, w2_ref[...], preferred_element_type=jnp.float32)
                  + b2_ref[...])


NR = N_PAD * C // 128


def _init_body(degp_ref, h_ref, dk_ref, u_ref, dis_ref):
    deg = degp_ref[0] + degp_ref[1]
    dis = jnp.where(deg > 0, lax.rsqrt(deg), 0.0)
    dis_ref[...] = dis
    u_ref[...] = dis * (dk_ref[0, 0] * h_ref[...])


def _combine_body(y_ref, dis_ref, h_ref, dp_ref, u_ref):
    dis = dis_ref[...]
    r = dp_ref[0, 0] * h_ref[...] - dis * (y_ref[0] + y_ref[1])
    u_ref[...] = dis * r


def _fin_body(y_ref, dis_ref, h_ref, d0_ref, r_ref):
    r_ref[...] = (d0_ref[0, 0] * h_ref[...]
                  - dis_ref[...] * (y_ref[0] + y_ref[1]))


def _softmax_body(r_ref, o_ref):
    r = r_ref[...]
    m = jnp.max(r, axis=1, keepdims=True)
    o_ref[...] = (r - m) - jnp.log(jnp.sum(jnp.exp(r - m), axis=1,
                                           keepdims=True))


_VMEM = pl.BlockSpec(memory_space=pltpu.VMEM)
_SMEM = pl.BlockSpec(memory_space=pltpu.SMEM)


def _shape(s):
    return jax.ShapeDtypeStruct(s, jnp.float32)


_mlp = pl.pallas_call(
    _mlp_body,
    in_specs=[_VMEM] * 5,
    out_specs=_VMEM,
    out_shape=_shape((N_PAD, C)),
)
_init = pl.pallas_call(
    _init_body,
    in_specs=[_VMEM, _VMEM, _SMEM],
    out_specs=(_VMEM, _VMEM),
    out_shape=(_shape((NR, 128)), _shape((NR, 128))),
)
_combine = pl.pallas_call(
    _combine_body,
    in_specs=[_VMEM, _VMEM, _VMEM, _SMEM],
    out_specs=_VMEM,
    out_shape=_shape((NR, 128)),
)
_fin = pl.pallas_call(
    _fin_body,
    in_specs=[_VMEM, _VMEM, _VMEM, _SMEM],
    out_specs=_VMEM,
    out_shape=_shape((NR, 128)),
)
_softmax = pl.pallas_call(
    _softmax_body,
    in_specs=[_VMEM],
    out_specs=_VMEM,
    out_shape=_shape((N_PAD, C)),
)


def kernel(x, edge_index, W1, b1, W2, b2, temp):
    f32 = jnp.float32
    row = edge_index[0]
    col = edge_index[1]
    npad = E_SCAT - E
    pad_r = (jnp.int32(N) + jnp.arange(npad, dtype=jnp.int32) % (N_PAD - N))
    pad_c = (jnp.int32(N)
             + (jnp.arange(npad, dtype=jnp.int32) * 7 + 3) % (N_PAD - N))
    nov = E_TOT - E_SCAT
    ov = ((jnp.arange(nov, dtype=jnp.int32) * 97) % N_PAD).reshape(NW, NOV, CH)
    src_arr = jnp.concatenate(
        [jnp.concatenate([row, pad_r]).reshape(NW, NCH, CH), ov], axis=1)
    dst_arr = jnp.concatenate(
        [jnp.concatenate([col, pad_c]).reshape(NW, NCH, CH), ov], axis=1)

    zeros_t = jnp.zeros((N_PAD, C), f32)
    ones_t = jnp.ones((N_PAD, C), f32)
    x_pad = jnp.zeros((N_PAD, D), f32).at[:N].set(x)

    h = _mlp(x_pad, W1, b1.reshape(1, HID), W2, b2.reshape(1, C))
    h_lin = h.reshape(NR, 128)
    dvec = (jax.nn.relu(temp) * _BINOM) @ _U

    sc_scatter_add = _get_sc_scatter_add()
    degp = sc_scatter_add(ones_t, dst_arr, src_arr, zeros_t)
    u, dis = _init(degp.reshape(NC, NR, 128), h_lin, dvec[K].reshape(1, 1))

    for p in range(K - 1, 0, -1):
        y = sc_scatter_add(u.reshape(N_PAD, C), src_arr, dst_arr, zeros_t)
        u = _combine(y.reshape(NC, NR, 128), dis, h_lin,
                     dvec[p].reshape(1, 1))
    y = sc_scatter_add(u.reshape(N_PAD, C), src_arr, dst_arr, zeros_t)
    r_lin = _fin(y.reshape(NC, NR, 128), dis, h_lin, dvec[0].reshape(1, 1))
    out = _softmax(r_lin.reshape(N_PAD, C))
    return out[:N]


# speedup vs baseline: 221.0419x; 1.0891x over previous
import functools
import math

import jax
import jax.numpy as jnp
import numpy as np
from jax import lax
from jax.experimental import pallas as pl
from jax.experimental.pallas import tpu as pltpu
from jax.experimental.pallas import tpu_sc as plsc

N = 10000
E = 320000
D = 128
HID = 64
C = 16
K = 10

NC = 2
NS = 16
NW = NC * NS
CH = 128
N_PAD = 10240
PER_TILE = E // NW
NCH = 80
NBUF = 8
NOV = NBUF
NCH_TOT = NCH + NOV
E_SCAT = NW * NCH * CH
E_TOT = NW * NCH_TOT * CH
ROWS_PER_TILE = N_PAD // NS


def _coef_matrix():
    u = np.zeros((K + 1, K + 1), dtype=np.float64)
    for j in range(K + 1):
        for a in range(j + 1):
            for b in range(K - j + 1):
                u[j, a + b] += math.comb(j, a) * math.comb(K - j, b) * ((-1.0) ** b)
    return u.astype(np.float32)


_U = _coef_matrix()
_BINOM = np.asarray([math.comb(K, j) / 2.0**K for j in range(K + 1)],
                    dtype=np.float32)

@functools.cache
def _get_sc_scatter_add():
    mesh = plsc.VectorSubcoreMesh(core_axis_name="c", subcore_axis_name="s")

    @functools.partial(
        pl.kernel,
        mesh=mesh,
        compiler_params=pltpu.CompilerParams(use_tc_tiling_on_sc=False),
        out_type=jax.ShapeDtypeStruct((NC, N_PAD, C), jnp.float32),
        scratch_types=[
            pltpu.VMEM((NCH_TOT, CH), jnp.int32),
            pltpu.VMEM((NCH_TOT, CH), jnp.int32),
            pltpu.VMEM((2, NBUF, CH, C), jnp.float32),
            pltpu.VMEM((ROWS_PER_TILE, C), jnp.float32),
            pltpu.VMEM_SHARED((N_PAD, C), jnp.float32),
            pltpu.SemaphoreType.DMA,
            pltpu.SemaphoreType.DMA,
        ],
    )
    def _sc_scatter_add(table, src, dst, zeros, out, src_v, dst_v, rows_v,
                        stage_v, acc, gsem, ssem):
        cid = lax.axis_index("c")
        sid = lax.axis_index("s")
        wid = cid * NS + sid
        pltpu.sync_copy(src.at[wid], src_v)
        pltpu.sync_copy(dst.at[wid], dst_v)
        r0 = sid * ROWS_PER_TILE
        pltpu.sync_copy(zeros.at[pl.ds(r0, ROWS_PER_TILE)], stage_v)
        pltpu.sync_copy(stage_v, acc.at[pl.ds(r0, ROWS_PER_TILE)])
        plsc.subcore_barrier()

        def issue_gathers(g, s):
            for b in range(NBUF):
                pltpu.async_copy(table.at[src_v.at[g * NBUF + b]],
                                 rows_v.at[s, b], gsem)

        def drain_gathers(s):
            for b in range(NBUF):
                pltpu.make_async_copy(table.at[src_v.at[0]],
                                      rows_v.at[s, b], gsem).wait()

        def issue_scatters(g, s):
            for b in range(NBUF):
                pltpu.async_copy(rows_v.at[s, b],
                                 acc.at[dst_v.at[g * NBUF + b]], ssem,
                                 add=True)

        def drain_scatters(s):
            for b in range(NBUF):
                pltpu.make_async_copy(rows_v.at[s, b],
                                      acc.at[dst_v.at[0]], ssem).wait()

        issue_gathers(0, 0)

        def body(g2, carry):
            ga = 2 * g2
            gb = 2 * g2 + 1
            drain_gathers(0)
            issue_gathers(gb, 1)
            issue_scatters(ga, 0)
            drain_gathers(1)
            drain_scatters(0)
            issue_gathers(ga + 2, 0)
            issue_scatters(gb, 1)
            drain_scatters(1)
            return carry

        lax.fori_loop(0, NCH // (2 * NBUF), body, 0)
        drain_gathers(0)
        plsc.subcore_barrier()
        pltpu.sync_copy(acc.at[pl.ds(r0, ROWS_PER_TILE)], stage_v)
        pltpu.sync_copy(stage_v, out.at[cid, pl.ds(r0, ROWS_PER_TILE)])

    return _sc_scatter_add


def _mlp_body(x_ref, w1_ref, b1_ref, w2_ref, b2_ref, h_ref):
    h1 = jnp.dot(x_ref[...], w1_ref[...], preferred_element_type=jnp.float32)
    h1 = jnp.maximum(h1 + b1_ref[...], 0.0)
    h_ref[...] = (jnp.dot(h1, w2_ref[...], preferred_element_type=jnp.float32)
                  + b2_ref[...])


NR = N_PAD * C // 128


def _init_body(degp_ref, h_ref, dk_ref, u_ref, dis_ref):
    deg = degp_ref[0] + degp_ref[1]
    dis = jnp.where(deg > 0, lax.rsqrt(deg), 0.0)
    dis_ref[...] = dis
    u_ref[...] = dis * (dk_ref[0, 0] * h_ref[...])


def _combine_body(y_ref, dis_ref, h_ref, dp_ref, u_ref):
    dis = dis_ref[...]
    r = dp_ref[0, 0] * h_ref[...] - dis * (y_ref[0] + y_ref[1])
    u_ref[...] = dis * r


def _fin_body(y_ref, dis_ref, h_ref, d0_ref, r_ref):
    r_ref[...] = (d0_ref[0, 0] * h_ref[...]
                  - dis_ref[...] * (y_ref[0] + y_ref[1]))


def _softmax_body(r_ref, o_ref):
    r = r_ref[...]
    m = jnp.max(r, axis=1, keepdims=True)
    o_ref[...] = (r - m) - jnp.log(jnp.sum(jnp.exp(r - m), axis=1,
                                           keepdims=True))


_VMEM = pl.BlockSpec(memory_space=pltpu.VMEM)
_SMEM = pl.BlockSpec(memory_space=pltpu.SMEM)


def _shape(s):
    return jax.ShapeDtypeStruct(s, jnp.float32)


_mlp = pl.pallas_call(
    _mlp_body,
    in_specs=[_VMEM] * 5,
    out_specs=_VMEM,
    out_shape=_shape((N_PAD, C)),
)
_init = pl.pallas_call(
    _init_body,
    in_specs=[_VMEM, _VMEM, _SMEM],
    out_specs=(_VMEM, _VMEM),
    out_shape=(_shape((NR, 128)), _shape((NR, 128))),
)
_combine = pl.pallas_call(
    _combine_body,
    in_specs=[_VMEM, _VMEM, _VMEM, _SMEM],
    out_specs=_VMEM,
    out_shape=_shape((NR, 128)),
)
_fin = pl.pallas_call(
    _fin_body,
    in_specs=[_VMEM, _VMEM, _VMEM, _SMEM],
    out_specs=_VMEM,
    out_shape=_shape((NR, 128)),
)
_softmax = pl.pallas_call(
    _softmax_body,
    in_specs=[_VMEM],
    out_specs=_VMEM,
    out_shape=_shape((N_PAD, C)),
)


def kernel(x, edge_index, W1, b1, W2, b2, temp):
    f32 = jnp.float32
    row = edge_index[0]
    col = edge_index[1]
    npad = E_SCAT - E
    pad_r = (jnp.int32(N) + jnp.arange(npad, dtype=jnp.int32) % (N_PAD - N))
    pad_c = (jnp.int32(N)
             + (jnp.arange(npad, dtype=jnp.int32) * 7 + 3) % (N_PAD - N))
    nov = E_TOT - E_SCAT
    ov = ((jnp.arange(nov, dtype=jnp.int32) * 97) % N_PAD).reshape(NW, NOV, CH)
    src_arr = jnp.concatenate(
        [jnp.concatenate([row, pad_r]).reshape(NW, NCH, CH), ov], axis=1)
    dst_arr = jnp.concatenate(
        [jnp.concatenate([col, pad_c]).reshape(NW, NCH, CH), ov], axis=1)

    zeros_t = jnp.zeros((N_PAD, C), f32)
    ones_t = jnp.ones((N_PAD, C), f32)
    x_pad = jnp.zeros((N_PAD, D), f32).at[:N].set(x)

    h = _mlp(x_pad, W1, b1.reshape(1, HID), W2, b2.reshape(1, C))
    h_lin = h.reshape(NR, 128)
    dvec = (jax.nn.relu(temp) * _BINOM) @ _U

    sc_scatter_add = _get_sc_scatter_add()
    degp = sc_scatter_add(ones_t, dst_arr, src_arr, zeros_t)
    u, dis = _init(degp.reshape(NC, NR, 128), h_lin, dvec[K].reshape(1, 1))

    for p in range(K - 1, 0, -1):
        y = sc_scatter_add(u.reshape(N_PAD, C), src_arr, dst_arr, zeros_t)
        u = _combine(y.reshape(NC, NR, 128), dis, h_lin,
                     dvec[p].reshape(1, 1))
    y = sc_scatter_add(u.reshape(N_PAD, C), src_arr, dst_arr, zeros_t)
    r_lin = _fin(y.reshape(NC, NR, 128), dis, h_lin, dvec[0].reshape(1, 1))
    out = _softmax(r_lin.reshape(N_PAD, C))
    return out[:N]


# speedup vs baseline: 230.9339x; 1.0448x over previous
import functools
import math

import jax
import jax.numpy as jnp
import numpy as np
from jax import lax
from jax.experimental import pallas as pl
from jax.experimental.pallas import tpu as pltpu
from jax.experimental.pallas import tpu_sc as plsc

N = 10000
E = 320000
D = 128
HID = 64
C = 16
K = 10

NC = 2
NS = 16
NW = NC * NS
CH = 128
N_PAD = 10240
PER_TILE = E // NW
NCH = 80
NBUF = 8
NOV = NBUF
NCH_TOT = NCH + NOV
E_SCAT = NW * NCH * CH
E_TOT = NW * NCH_TOT * CH
ROWS_PER_TILE = N_PAD // NS


def _coef_matrix():
    u = np.zeros((K + 1, K + 1), dtype=np.float64)
    for j in range(K + 1):
        for a in range(j + 1):
            for b in range(K - j + 1):
                u[j, a + b] += math.comb(j, a) * math.comb(K - j, b) * ((-1.0) ** b)
    return u.astype(np.float32)


_U = _coef_matrix()
_BINOM = np.asarray([math.comb(K, j) / 2.0**K for j in range(K + 1)],
                    dtype=np.float32)

@functools.cache
def _get_sc_scatter_add():
    mesh = plsc.VectorSubcoreMesh(core_axis_name="c", subcore_axis_name="s")

    @functools.partial(
        pl.kernel,
        mesh=mesh,
        compiler_params=pltpu.CompilerParams(use_tc_tiling_on_sc=False),
        out_type=jax.ShapeDtypeStruct((NC, N_PAD, C), jnp.float32),
        scratch_types=[
            pltpu.VMEM((NCH_TOT, CH), jnp.int32),
            pltpu.VMEM((NCH_TOT, CH), jnp.int32),
            pltpu.VMEM((2, NBUF, CH, C), jnp.float32),
            pltpu.VMEM((ROWS_PER_TILE, C), jnp.float32),
            pltpu.VMEM_SHARED((N_PAD, C), jnp.float32),
            pltpu.SemaphoreType.DMA,
            pltpu.SemaphoreType.DMA,
        ],
    )
    def _sc_scatter_add(table, src, dst, zeros, out, src_v, dst_v, rows_v,
                        stage_v, acc, gsem, ssem):
        cid = lax.axis_index("c")
        sid = lax.axis_index("s")
        wid = cid * NS + sid
        pltpu.sync_copy(src.at[wid], src_v)
        pltpu.sync_copy(dst.at[wid], dst_v)
        r0 = sid * ROWS_PER_TILE
        pltpu.sync_copy(zeros.at[pl.ds(r0, ROWS_PER_TILE)], stage_v)
        pltpu.sync_copy(stage_v, acc.at[pl.ds(r0, ROWS_PER_TILE)])
        plsc.subcore_barrier()

        def issue_gathers(g, s):
            for b in range(NBUF):
                pltpu.async_copy(table.at[src_v.at[g * NBUF + b]],
                                 rows_v.at[s, b], gsem)

        def drain_gathers(s):
            for b in range(NBUF):
                pltpu.make_async_copy(table.at[src_v.at[0]],
                                      rows_v.at[s, b], gsem).wait()

        def issue_scatters(g, s):
            for b in range(NBUF):
                pltpu.async_copy(rows_v.at[s, b],
                                 acc.at[dst_v.at[g * NBUF + b]], ssem,
                                 add=True)

        def drain_scatters(s):
            for b in range(NBUF):
                pltpu.make_async_copy(rows_v.at[s, b],
                                      acc.at[dst_v.at[0]], ssem).wait()

        issue_gathers(0, 0)

        def body(g2, carry):
            ga = 2 * g2
            gb = 2 * g2 + 1
            drain_gathers(0)
            issue_gathers(gb, 1)
            issue_scatters(ga, 0)
            drain_gathers(1)
            drain_scatters(0)
            issue_gathers(ga + 2, 0)
            issue_scatters(gb, 1)
            drain_scatters(1)
            return carry

        lax.fori_loop(0, NCH // (2 * NBUF), body, 0)
        drain_gathers(0)
        plsc.subcore_barrier()
        pltpu.sync_copy(acc.at[pl.ds(r0, ROWS_PER_TILE)], stage_v)
        pltpu.sync_copy(stage_v, out.at[cid, pl.ds(r0, ROWS_PER_TILE)])

    return _sc_scatter_add


@functools.cache
def _get_sc_degree():
    mesh = plsc.VectorSubcoreMesh(core_axis_name="c", subcore_axis_name="s")

    @functools.partial(
        pl.kernel,
        mesh=mesh,
        compiler_params=pltpu.CompilerParams(use_tc_tiling_on_sc=False),
        out_type=jax.ShapeDtypeStruct((NC, N_PAD, C), jnp.float32),
        scratch_types=[
            pltpu.VMEM((NCH_TOT, CH), jnp.int32),
            pltpu.VMEM((CH, C), jnp.float32),
            pltpu.VMEM((ROWS_PER_TILE, C), jnp.float32),
            pltpu.VMEM_SHARED((N_PAD, C), jnp.float32),
            pltpu.SemaphoreType.DMA,
        ],
    )
    def _sc_degree(idx, ones_small, zeros, out, idx_v, ones_v, stage_v, acc,
                   ssem):
        cid = lax.axis_index("c")
        sid = lax.axis_index("s")
        wid = cid * NS + sid
        pltpu.sync_copy(idx.at[wid], idx_v)
        r0 = sid * ROWS_PER_TILE
        pltpu.sync_copy(zeros.at[pl.ds(r0, ROWS_PER_TILE)], stage_v)
        pltpu.sync_copy(stage_v, acc.at[pl.ds(r0, ROWS_PER_TILE)])
        pltpu.sync_copy(ones_small, ones_v)
        plsc.subcore_barrier()

        def issue(g):
            for b in range(NBUF):
                pltpu.async_copy(ones_v, acc.at[idx_v.at[g * NBUF + b]],
                                 ssem, add=True)

        def drain_group():
            for b in range(NBUF):
                pltpu.make_async_copy(ones_v, acc.at[idx_v.at[0]],
                                      ssem).wait()

        issue(0)

        def body(g, carry):
            issue(g)
            drain_group()
            return carry

        lax.fori_loop(1, NCH // NBUF, body, 0)
        drain_group()
        plsc.subcore_barrier()
        pltpu.sync_copy(acc.at[pl.ds(r0, ROWS_PER_TILE)], stage_v)
        pltpu.sync_copy(stage_v, out.at[cid, pl.ds(r0, ROWS_PER_TILE)])

    return _sc_degree


def _mlp_body(x_ref, w1_ref, b1_ref, w2_ref, b2_ref, h_ref):
    h1 = jnp.dot(x_ref[...], w1_ref[...], preferred_element_type=jnp.float32)
    h1 = jnp.maximum(h1 + b1_ref[...], 0.0)
    h_ref[...] = (jnp.dot(h1, w2_ref[...], preferred_element_type=jnp.float32)
                  + b2_ref[...])


NR = N_PAD * C // 128


def _init_body(degp_ref, h_ref, dk_ref, u_ref, dis_ref):
    deg = degp_ref[0] + degp_ref[1]
    dis = jnp.where(deg > 0, lax.rsqrt(deg), 0.0)
    dis_ref[...] = dis
    u_ref[...] = dis * (dk_ref[0, 0] * h_ref[...])


def _combine_body(y_ref, dis_ref, h_ref, dp_ref, u_ref):
    dis = dis_ref[...]
    r = dp_ref[0, 0] * h_ref[...] - dis * (y_ref[0] + y_ref[1])
    u_ref[...] = dis * r


def _fin_body(y_ref, dis_ref, h_ref, d0_ref, r_ref):
    r_ref[...] = (d0_ref[0, 0] * h_ref[...]
                  - dis_ref[...] * (y_ref[0] + y_ref[1]))


def _softmax_body(r_ref, o_ref):
    r = r_ref[...]
    m = jnp.max(r, axis=1, keepdims=True)
    o_ref[...] = (r - m) - jnp.log(jnp.sum(jnp.exp(r - m), axis=1,
                                           keepdims=True))


_VMEM = pl.BlockSpec(memory_space=pltpu.VMEM)
_SMEM = pl.BlockSpec(memory_space=pltpu.SMEM)


def _shape(s):
    return jax.ShapeDtypeStruct(s, jnp.float32)


_mlp = pl.pallas_call(
    _mlp_body,
    in_specs=[_VMEM] * 5,
    out_specs=_VMEM,
    out_shape=_shape((N_PAD, C)),
)
_init = pl.pallas_call(
    _init_body,
    in_specs=[_VMEM, _VMEM, _SMEM],
    out_specs=(_VMEM, _VMEM),
    out_shape=(_shape((NR, 128)), _shape((NR, 128))),
)
_combine = pl.pallas_call(
    _combine_body,
    in_specs=[_VMEM, _VMEM, _VMEM, _SMEM],
    out_specs=_VMEM,
    out_shape=_shape((NR, 128)),
)
_fin = pl.pallas_call(
    _fin_body,
    in_specs=[_VMEM, _VMEM, _VMEM, _SMEM],
    out_specs=_VMEM,
    out_shape=_shape((NR, 128)),
)
_softmax = pl.pallas_call(
    _softmax_body,
    in_specs=[_VMEM],
    out_specs=_VMEM,
    out_shape=_shape((N_PAD, C)),
)


def kernel(x, edge_index, W1, b1, W2, b2, temp):
    f32 = jnp.float32
    row = edge_index[0]
    col = edge_index[1]
    npad = E_SCAT - E
    pad_r = (jnp.int32(N) + jnp.arange(npad, dtype=jnp.int32) % (N_PAD - N))
    pad_c = (jnp.int32(N)
             + (jnp.arange(npad, dtype=jnp.int32) * 7 + 3) % (N_PAD - N))
    nov = E_TOT - E_SCAT
    ov = ((jnp.arange(nov, dtype=jnp.int32) * 97) % N_PAD).reshape(NW, NOV, CH)
    src_arr = jnp.concatenate(
        [jnp.concatenate([row, pad_r]).reshape(NW, NCH, CH), ov], axis=1)
    dst_arr = jnp.concatenate(
        [jnp.concatenate([col, pad_c]).reshape(NW, NCH, CH), ov], axis=1)

    zeros_t = jnp.zeros((N_PAD, C), f32)
    ones_small = jnp.ones((CH, C), f32)
    x_pad = jnp.zeros((N_PAD, D), f32).at[:N].set(x)

    h = _mlp(x_pad, W1, b1.reshape(1, HID), W2, b2.reshape(1, C))
    h_lin = h.reshape(NR, 128)
    dvec = (jax.nn.relu(temp) * _BINOM) @ _U

    sc_scatter_add = _get_sc_scatter_add()
    degp = _get_sc_degree()(src_arr, ones_small, zeros_t)
    u, dis = _init(degp.reshape(NC, NR, 128), h_lin, dvec[K].reshape(1, 1))

    for p in range(K - 1, 0, -1):
        y = sc_scatter_add(u.reshape(N_PAD, C), src_arr, dst_arr, zeros_t)
        u = _combine(y.reshape(NC, NR, 128), dis, h_lin,
                     dvec[p].reshape(1, 1))
    y = sc_scatter_add(u.reshape(N_PAD, C), src_arr, dst_arr, zeros_t)
    r_lin = _fin(y.reshape(NC, NR, 128), dis, h_lin, dvec[0].reshape(1, 1))
    out = _softmax(r_lin.reshape(N_PAD, C))
    return out[:N]
